# baseline (speedup 1.0000x reference)
import jax
import jax.numpy as jnp
from jax.experimental import pallas as pl
from jax.experimental.pallas import tpu as pltpu

N = 5000
ROWS = 40
LANES = 128
NPAD = ROWS * LANES
BB_NUM = 100
THR = 0.5
CLIP_MAX = 511.0
B = 4


def _nms_kernel(c0_ref, c1_ref, r0_ref, r1_ref, r2_ref, r3_ref,
                a0_ref, a1_ref, a2_ref, a3_ref, out_ref,
                sc_ref, x0_ref, y0_ref, x1_ref, y1_ref, ar_ref):
    flat = (jax.lax.broadcasted_iota(jnp.int32, (ROWS, LANES), 0) * LANES
            + jax.lax.broadcasted_iota(jnp.int32, (ROWS, LANES), 1))
    pad = flat >= N
    lane1 = jax.lax.broadcasted_iota(jnp.int32, (1, LANES), 1)
    olane = lane1

    for b in range(B):
        e0 = jnp.exp(c0_ref[b])
        e1 = jnp.exp(c1_ref[b])
        sc_ref[b] = jnp.where(pad, -1.0, e0 / (e0 + e1))
        xmin = jnp.maximum(a0_ref[...] - r0_ref[b], 0.0)
        ymin = jnp.maximum(a1_ref[...] - r1_ref[b], 0.0)
        xmax = jnp.minimum(a2_ref[...] + r2_ref[b], CLIP_MAX)
        ymax = jnp.minimum(a3_ref[...] + r3_ref[b], CLIP_MAX)
        x0_ref[b] = xmin
        y0_ref[b] = ymin
        x1_ref[b] = xmax
        y1_ref[b] = ymax
        ar_ref[b] = (jnp.maximum(xmax - xmin, 0.0)
                     * jnp.maximum(ymax - ymin, 0.0))

    zrow = jnp.zeros((1, LANES), jnp.float32)

    def body(t, carry):
        obs, lasts = carry
        new_obs, new_lasts = [], []
        for b in range(B):
            scb = sc_ref[b]
            m = jnp.max(scb)
            valid = m >= 0.0
            sel = jnp.min(jnp.where(scb == m, flat, NPAD))
            r = jnp.minimum(sel // LANES, ROWS - 1)
            l = sel % LANES
            lmask = lane1 == l
            bx0 = jnp.sum(jnp.where(lmask, x0_ref[b, pl.ds(r, 1), :], 0.0))
            bx1 = jnp.sum(jnp.where(lmask, y0_ref[b, pl.ds(r, 1), :], 0.0))
            bx2 = jnp.sum(jnp.where(lmask, x1_ref[b, pl.ds(r, 1), :], 0.0))
            bx3 = jnp.sum(jnp.where(lmask, y1_ref[b, pl.ds(r, 1), :], 0.0))
            selarea = (jnp.maximum(bx2 - bx0, 0.0)
                       * jnp.maximum(bx3 - bx1, 0.0))
            xx1 = jnp.maximum(x0_ref[b], bx0)
            yy1 = jnp.maximum(y0_ref[b], bx1)
            xx2 = jnp.minimum(x1_ref[b], bx2)
            yy2 = jnp.minimum(y1_ref[b], bx3)
            inter = (jnp.maximum(xx2 - xx1, 0.0)
                     * jnp.maximum(yy2 - yy1, 0.0))
            iou = inter / (ar_ref[b] + selarea - inter + 1e-9)
            self_m = (scb == m) & (flat == sel)
            sc_ref[b] = jnp.where((iou > THR) | self_m, -1.0, scb)
            l0, l1, l2, l3, ls = lasts[b]
            v0 = jnp.where(valid, bx0, l0)
            v1 = jnp.where(valid, bx1, l1)
            v2 = jnp.where(valid, bx2, l2)
            v3 = jnp.where(valid, bx3, l3)
            vs = jnp.where(valid, m, ls)
            tm = olane == t
            ob0, ob1, ob2, ob3, osc = obs[b]
            new_obs.append((jnp.where(tm, v0, ob0), jnp.where(tm, v1, ob1),
                            jnp.where(tm, v2, ob2), jnp.where(tm, v3, ob3),
                            jnp.where(tm, vs, osc)))
            new_lasts.append((v0, v1, v2, v3, vs))
        return (tuple(new_obs), tuple(new_lasts))

    init = (tuple((zrow,) * 5 for _ in range(B)),
            tuple((0.0,) * 5 for _ in range(B)))
    obs, _ = jax.lax.fori_loop(0, BB_NUM, body, init)
    for b in range(B):
        out_ref[b] = jnp.concatenate(list(obs[b]) + [zrow, zrow, zrow],
                                     axis=0)


def kernel(cl, re, anc):
    pad = NPAD - N

    def prep(x):
        return jnp.pad(x, ((0, 0), (0, pad))).reshape(B, ROWS, LANES)

    def prepa(x):
        return jnp.pad(x, (0, pad)).reshape(ROWS, LANES)

    c0 = prep(cl[..., 0])
    c1 = prep(cl[..., 1])
    r0 = prep(re[..., 0])
    r1 = prep(re[..., 1])
    r2 = prep(re[..., 2])
    r3 = prep(re[..., 3])
    a0 = prepa(anc[0, :, 0])
    a1 = prepa(anc[0, :, 1])
    a2 = prepa(anc[0, :, 2])
    a3 = prepa(anc[0, :, 3])

    scratch = [pltpu.VMEM((B, ROWS, LANES), jnp.float32)] * 6
    out = pl.pallas_call(
        _nms_kernel,
        out_shape=jax.ShapeDtypeStruct((B, 8, LANES), jnp.float32),
        scratch_shapes=scratch,
    )(c0, c1, r0, r1, r2, r3, a0, a1, a2, a3)

    bb = jnp.stack([out[:, 0, :BB_NUM], out[:, 1, :BB_NUM],
                    out[:, 2, :BB_NUM], out[:, 3, :BB_NUM]], axis=-1)
    ffo = out[:, 4, :BB_NUM]
    return bb, ffo


# speedup vs baseline: 3.1439x; 3.1439x over previous
import jax
import jax.numpy as jnp
from jax.experimental import pallas as pl

N = 5000
ROWS = 40
LANES = 128
NPAD = ROWS * LANES
BB_NUM = 100
THR = 0.5
CLIP_MAX = 511.0
B = 4


def _nms_kernel(c0_ref, c1_ref, r0_ref, r1_ref, r2_ref, r3_ref,
                a0_ref, a1_ref, a2_ref, a3_ref, out_ref):
    e0 = jnp.exp(c0_ref[...])
    e1 = jnp.exp(c1_ref[...])
    ff = e0 / (e0 + e1)
    xmin = jnp.maximum(a0_ref[...][None] - r0_ref[...], 0.0)
    ymin = jnp.maximum(a1_ref[...][None] - r1_ref[...], 0.0)
    xmax = jnp.minimum(a2_ref[...][None] + r2_ref[...], CLIP_MAX)
    ymax = jnp.minimum(a3_ref[...][None] + r3_ref[...], CLIP_MAX)
    areas = jnp.maximum(xmax - xmin, 0.0) * jnp.maximum(ymax - ymin, 0.0)
    row = jax.lax.broadcasted_iota(jnp.int32, (B, ROWS, LANES), 1)
    lane = jax.lax.broadcasted_iota(jnp.int32, (B, ROWS, LANES), 2)
    flat = row * LANES + lane
    olane = jax.lax.broadcasted_iota(jnp.int32, (B, 1, LANES), 2)
    zout = jnp.zeros((B, 1, LANES), jnp.float32)
    zsc = jnp.zeros((B, 1, 1), jnp.float32)
    zi = jnp.zeros((B, 1, 1), jnp.int32)

    def red_max(x):
        return jnp.max(x, axis=(1, 2), keepdims=True)

    def red_min(x):
        return jnp.min(x, axis=(1, 2), keepdims=True)

    def red_sum(x):
        return jnp.sum(x, axis=(1, 2), keepdims=True)

    msc0 = jnp.where(flat >= N, -1.0, ff)

    def cond(carry):
        cnt, alive = carry[0], carry[1]
        return jnp.any((cnt < BB_NUM) & (alive > 0))

    def body(carry):
        (cnt, alive, msc, ob0, ob1, ob2, ob3, osc,
         l0, l1, l2, l3, ls) = carry
        m1 = red_max(msc)
        valid1 = m1 >= 0.0
        sel1 = red_min(jnp.where(msc == m1, flat, NPAD))
        hit1 = flat == sel1
        p0 = red_sum(jnp.where(hit1, xmin, 0.0))
        p1 = red_sum(jnp.where(hit1, ymin, 0.0))
        p2 = red_sum(jnp.where(hit1, xmax, 0.0))
        p3 = red_sum(jnp.where(hit1, ymax, 0.0))
        msc2 = jnp.where(hit1, -1.0, msc)
        m2 = red_max(msc2)
        valid2 = m2 >= 0.0
        sel2 = red_min(jnp.where(msc2 == m2, flat, NPAD))
        hit2 = flat == sel2
        q0 = red_sum(jnp.where(hit2, xmin, 0.0))
        q1 = red_sum(jnp.where(hit2, ymin, 0.0))
        q2 = red_sum(jnp.where(hit2, xmax, 0.0))
        q3 = red_sum(jnp.where(hit2, ymax, 0.0))
        parea = jnp.maximum(p2 - p0, 0.0) * jnp.maximum(p3 - p1, 0.0)
        qarea = jnp.maximum(q2 - q0, 0.0) * jnp.maximum(q3 - q1, 0.0)
        i12 = (jnp.maximum(jnp.minimum(p2, q2) - jnp.maximum(p0, q0), 0.0)
               * jnp.maximum(jnp.minimum(p3, q3) - jnp.maximum(p1, q1), 0.0))
        iou12 = i12 / (parea + qarea - i12 + 1e-9)
        kept2 = valid1 & valid2 & jnp.logical_not(iou12 > THR)
        xx1 = jnp.maximum(xmin, p0)
        yy1 = jnp.maximum(ymin, p1)
        xx2 = jnp.minimum(xmax, p2)
        yy2 = jnp.minimum(ymax, p3)
        int1 = jnp.maximum(xx2 - xx1, 0.0) * jnp.maximum(yy2 - yy1, 0.0)
        iou1 = int1 / (areas + parea - int1 + 1e-9)
        ux1 = jnp.maximum(xmin, q0)
        uy1 = jnp.maximum(ymin, q1)
        ux2 = jnp.minimum(xmax, q2)
        uy2 = jnp.minimum(ymax, q3)
        int2 = jnp.maximum(ux2 - ux1, 0.0) * jnp.maximum(uy2 - uy1, 0.0)
        iou2 = int2 / (areas + qarea - int2 + 1e-9)
        sup = ((valid1 & ((iou1 > THR) | hit1))
               | (kept2 & ((iou2 > THR) | hit2)))
        msc = jnp.where(sup, -1.0, msc)
        tm1 = (olane == cnt) & valid1
        ob0 = jnp.where(tm1, p0, ob0)
        ob1 = jnp.where(tm1, p1, ob1)
        ob2 = jnp.where(tm1, p2, ob2)
        ob3 = jnp.where(tm1, p3, ob3)
        osc = jnp.where(tm1, m1, osc)
        c2 = cnt + jnp.where(valid1, 1, 0)
        tm2 = (olane == c2) & kept2
        ob0 = jnp.where(tm2, q0, ob0)
        ob1 = jnp.where(tm2, q1, ob1)
        ob2 = jnp.where(tm2, q2, ob2)
        ob3 = jnp.where(tm2, q3, ob3)
        osc = jnp.where(tm2, m2, osc)
        l0 = jnp.where(kept2, q0, jnp.where(valid1, p0, l0))
        l1 = jnp.where(kept2, q1, jnp.where(valid1, p1, l1))
        l2 = jnp.where(kept2, q2, jnp.where(valid1, p2, l2))
        l3 = jnp.where(kept2, q3, jnp.where(valid1, p3, l3))
        ls = jnp.where(kept2, m2, jnp.where(valid1, m1, ls))
        cnt = c2 + jnp.where(kept2, 1, 0)
        alive = jnp.where(valid1, 1, 0)
        return (cnt, alive, msc, ob0, ob1, ob2, ob3, osc,
                l0, l1, l2, l3, ls)

    init = (zi, zi + 1, msc0, zout, zout, zout, zout, zout,
            zsc, zsc, zsc, zsc, zsc)
    res = jax.lax.while_loop(cond, body, init)
    cnt, ob0, ob1, ob2, ob3, osc = res[0], res[3], res[4], res[5], res[6], res[7]
    l0, l1, l2, l3, ls = res[8], res[9], res[10], res[11], res[12]
    fill = olane >= cnt
    ob0 = jnp.where(fill, l0, ob0)
    ob1 = jnp.where(fill, l1, ob1)
    ob2 = jnp.where(fill, l2, ob2)
    ob3 = jnp.where(fill, l3, ob3)
    osc = jnp.where(fill, ls, osc)
    zrow = jnp.zeros((B, 1, LANES), jnp.float32)
    out_ref[...] = jnp.concatenate(
        [ob0, ob1, ob2, ob3, osc, zrow, zrow, zrow], axis=1)


def kernel(cl, re, anc):
    pad = NPAD - N

    def prep(x):
        return jnp.pad(x, ((0, 0), (0, pad))).reshape(B, ROWS, LANES)

    def prepa(x):
        return jnp.pad(x, (0, pad)).reshape(ROWS, LANES)

    c0 = prep(cl[..., 0])
    c1 = prep(cl[..., 1])
    r0 = prep(re[..., 0])
    r1 = prep(re[..., 1])
    r2 = prep(re[..., 2])
    r3 = prep(re[..., 3])
    a0 = prepa(anc[0, :, 0])
    a1 = prepa(anc[0, :, 1])
    a2 = prepa(anc[0, :, 2])
    a3 = prepa(anc[0, :, 3])

    out = pl.pallas_call(
        _nms_kernel,
        out_shape=jax.ShapeDtypeStruct((B, 8, LANES), jnp.float32),
    )(c0, c1, r0, r1, r2, r3, a0, a1, a2, a3)

    bb = jnp.stack([out[:, 0, :BB_NUM], out[:, 1, :BB_NUM],
                    out[:, 2, :BB_NUM], out[:, 3, :BB_NUM]], axis=-1)
    ffo = out[:, 4, :BB_NUM]
    return bb, ffo


# speedup vs baseline: 3.3259x; 1.0579x over previous
import jax
import jax.numpy as jnp
from jax.experimental import pallas as pl

N = 5000
ROWS = 40
LANES = 128
NPAD = ROWS * LANES
BB_NUM = 100
THR = 0.5
CLIP_MAX = 511.0
B = 4
K = 4


def _nms_kernel(c0_ref, c1_ref, r0_ref, r1_ref, r2_ref, r3_ref,
                a0_ref, a1_ref, a2_ref, a3_ref, out_ref):
    e0 = jnp.exp(c0_ref[...])
    e1 = jnp.exp(c1_ref[...])
    ff = e0 / (e0 + e1)
    xmin = jnp.maximum(a0_ref[...][None] - r0_ref[...], 0.0)
    ymin = jnp.maximum(a1_ref[...][None] - r1_ref[...], 0.0)
    xmax = jnp.minimum(a2_ref[...][None] + r2_ref[...], CLIP_MAX)
    ymax = jnp.minimum(a3_ref[...][None] + r3_ref[...], CLIP_MAX)
    areas = jnp.maximum(xmax - xmin, 0.0) * jnp.maximum(ymax - ymin, 0.0)
    row = jax.lax.broadcasted_iota(jnp.int32, (B, ROWS, LANES), 1)
    lane = jax.lax.broadcasted_iota(jnp.int32, (B, ROWS, LANES), 2)
    flat = row * LANES + lane
    olane = jax.lax.broadcasted_iota(jnp.int32, (B, 1, LANES), 2)
    zout = jnp.zeros((B, 1, LANES), jnp.float32)
    zsc = jnp.zeros((B, 1, 1), jnp.float32)
    zi = jnp.zeros((B, 1, 1), jnp.int32)

    def red_max(x):
        return jnp.max(x, axis=(1, 2), keepdims=True)

    def red_min(x):
        return jnp.min(x, axis=(1, 2), keepdims=True)

    def red_sum(x):
        return jnp.sum(x, axis=(1, 2), keepdims=True)

    def pair_iou(a, b):
        iw = jnp.minimum(a[2], b[2]) - jnp.maximum(a[0], b[0])
        ih = jnp.minimum(a[3], b[3]) - jnp.maximum(a[1], b[1])
        inter = jnp.maximum(iw, 0.0) * jnp.maximum(ih, 0.0)
        return inter / (a[5] + b[5] - inter + 1e-9)

    msc0 = jnp.where(flat >= N, -1.0, ff)

    def cond(carry):
        cnt, alive = carry[0], carry[1]
        return jnp.any((cnt < BB_NUM) & (alive > 0))

    def body(carry):
        (cnt, alive, msc, ob0, ob1, ob2, ob3, osc,
         l0, l1, l2, l3, ls) = carry
        cands = []
        cur = msc
        for _ in range(K):
            m = red_max(cur)
            valid = m >= 0.0
            sel = red_min(jnp.where(cur == m, flat, NPAD))
            hit = flat == sel
            p0 = red_sum(jnp.where(hit, xmin, 0.0))
            p1 = red_sum(jnp.where(hit, ymin, 0.0))
            p2 = red_sum(jnp.where(hit, xmax, 0.0))
            p3 = red_sum(jnp.where(hit, ymax, 0.0))
            parea = jnp.maximum(p2 - p0, 0.0) * jnp.maximum(p3 - p1, 0.0)
            cands.append([p0, p1, p2, p3, m, parea, valid, hit])
            cur = jnp.where(hit, -1.0, cur)
        kept = [cands[0][6]]
        for k in range(1, K):
            alive_k = cands[k][6]
            for j in range(k):
                alive_k = alive_k & jnp.logical_not(
                    kept[j] & (pair_iou(cands[j], cands[k]) > THR))
            kept.append(alive_k)
        sup = None
        for k in range(K):
            p0, p1, p2, p3, m, parea, valid, hit = cands[k]
            xx1 = jnp.maximum(xmin, p0)
            yy1 = jnp.maximum(ymin, p1)
            xx2 = jnp.minimum(xmax, p2)
            yy2 = jnp.minimum(ymax, p3)
            inter = jnp.maximum(xx2 - xx1, 0.0) * jnp.maximum(yy2 - yy1, 0.0)
            iou = inter / (areas + parea - inter + 1e-9)
            s = kept[k] & ((iou > THR) | hit)
            sup = s if sup is None else (sup | s)
        msc = jnp.where(sup, -1.0, msc)
        pos = cnt
        for k in range(K):
            p0, p1, p2, p3, m, parea, valid, hit = cands[k]
            tm = (olane == pos) & kept[k]
            ob0 = jnp.where(tm, p0, ob0)
            ob1 = jnp.where(tm, p1, ob1)
            ob2 = jnp.where(tm, p2, ob2)
            ob3 = jnp.where(tm, p3, ob3)
            osc = jnp.where(tm, m, osc)
            l0 = jnp.where(kept[k], p0, l0)
            l1 = jnp.where(kept[k], p1, l1)
            l2 = jnp.where(kept[k], p2, l2)
            l3 = jnp.where(kept[k], p3, l3)
            ls = jnp.where(kept[k], m, ls)
            pos = pos + jnp.where(kept[k], 1, 0)
        cnt = pos
        alive = jnp.where(cands[0][6], 1, 0)
        return (cnt, alive, msc, ob0, ob1, ob2, ob3, osc,
                l0, l1, l2, l3, ls)

    init = (zi, zi + 1, msc0, zout, zout, zout, zout, zout,
            zsc, zsc, zsc, zsc, zsc)
    res = jax.lax.while_loop(cond, body, init)
    cnt, ob0, ob1, ob2, ob3, osc = res[0], res[3], res[4], res[5], res[6], res[7]
    l0, l1, l2, l3, ls = res[8], res[9], res[10], res[11], res[12]
    fill = olane >= cnt
    ob0 = jnp.where(fill, l0, ob0)
    ob1 = jnp.where(fill, l1, ob1)
    ob2 = jnp.where(fill, l2, ob2)
    ob3 = jnp.where(fill, l3, ob3)
    osc = jnp.where(fill, ls, osc)
    zrow = jnp.zeros((B, 1, LANES), jnp.float32)
    out_ref[...] = jnp.concatenate(
        [ob0, ob1, ob2, ob3, osc, zrow, zrow, zrow], axis=1)


def kernel(cl, re, anc):
    pad = NPAD - N

    def prep(x):
        return jnp.pad(x, ((0, 0), (0, pad))).reshape(B, ROWS, LANES)

    def prepa(x):
        return jnp.pad(x, (0, pad)).reshape(ROWS, LANES)

    c0 = prep(cl[..., 0])
    c1 = prep(cl[..., 1])
    r0 = prep(re[..., 0])
    r1 = prep(re[..., 1])
    r2 = prep(re[..., 2])
    r3 = prep(re[..., 3])
    a0 = prepa(anc[0, :, 0])
    a1 = prepa(anc[0, :, 1])
    a2 = prepa(anc[0, :, 2])
    a3 = prepa(anc[0, :, 3])

    out = pl.pallas_call(
        _nms_kernel,
        out_shape=jax.ShapeDtypeStruct((B, 8, LANES), jnp.float32),
    )(c0, c1, r0, r1, r2, r3, a0, a1, a2, a3)

    bb = jnp.stack([out[:, 0, :BB_NUM], out[:, 1, :BB_NUM],
                    out[:, 2, :BB_NUM], out[:, 3, :BB_NUM]], axis=-1)
    ffo = out[:, 4, :BB_NUM]
    return bb, ffo
